# baseline (speedup 1.0000x reference)
import functools

import jax
import jax.numpy as jnp
from jax import lax
from jax.experimental import pallas as pl
from jax.experimental.pallas import tpu as pltpu
from jax.experimental.pallas import tpu_sc as plsc

EMB = 64
NC, NS = 2, 16
NW = NC * NS
CHUNK = 100
NCH = 4
SUPER = CHUNK * NCH


@functools.cache
def _make_gather(B: int):
    bpw = B // NW
    rpw = bpw // 2
    nsuper = rpw // SUPER
    mesh = plsc.VectorSubcoreMesh(core_axis_name="c", subcore_axis_name="s")

    @functools.partial(
        pl.kernel,
        out_type=jax.ShapeDtypeStruct((B // 2, 2 * EMB), jnp.float32),
        mesh=mesh,
        compiler_params=pltpu.CompilerParams(use_tc_tiling_on_sc=False),
        scratch_types=[
            pltpu.VMEM((nsuper, 2, NCH, CHUNK), jnp.int32),
            pltpu.VMEM((2, SUPER, EMB), jnp.float32),
            pltpu.VMEM((2, SUPER, EMB), jnp.float32),
            pltpu.SemaphoreType.DMA,
            pltpu.SemaphoreType.DMA,
            pltpu.SemaphoreType.DMA,
            pltpu.SemaphoreType.DMA,
        ],
    )
    def gather_kernel(idx_hbm, table_hbm, out_hbm, idx_v, buf_a, buf_b,
                      gs_a, gs_b, os_a, os_b):
        wid = lax.axis_index("s") * NC + lax.axis_index("c")
        base = wid * rpw
        pltpu.sync_copy(idx_hbm.at[wid], idx_v)

        def start_gathers(s, buf, sem):
            for h in range(2):
                for c in range(NCH):
                    pltpu.async_copy(
                        table_hbm.at[idx_v.at[s, h, c]],
                        buf.at[h, pl.ds(c * CHUNK, CHUNK)], sem)

        def wait_gathers(s, buf, sem):
            for h in range(2):
                for c in range(NCH):
                    pltpu.make_async_copy(
                        table_hbm.at[idx_v.at[s, h, c]],
                        buf.at[h, pl.ds(c * CHUNK, CHUNK)], sem).wait()

        def out_descs(s, buf, sem):
            return [
                pltpu.make_async_copy(
                    buf.at[h],
                    out_hbm.at[pl.ds(base + s * SUPER, SUPER),
                               pl.ds(h * EMB, EMB)], sem)
                for h in range(2)
            ]

        def out_start(s, buf, sem):
            for d in out_descs(s, buf, sem):
                d.start()

        def out_wait(s, buf, sem):
            for d in out_descs(s, buf, sem):
                d.wait()

        start_gathers(0, buf_a, gs_a)

        def body(it, _):
            s0 = it * 2
            s1 = s0 + 1
            wait_gathers(s0, buf_a, gs_a)

            @pl.when(it > 0)
            def _():
                out_wait(s1 - 2, buf_b, os_b)

            start_gathers(s1, buf_b, gs_b)
            out_start(s0, buf_a, os_a)
            wait_gathers(s1, buf_b, gs_b)
            out_wait(s0, buf_a, os_a)

            @pl.when(s0 + 2 < nsuper)
            def _():
                start_gathers(s0 + 2, buf_a, gs_a)

            out_start(s1, buf_b, os_b)
            return ()

        lax.fori_loop(0, nsuper // 2, body, (), unroll=False)
        out_wait(nsuper - 1, buf_b, os_b)

    return gather_kernel


def kernel(multi_hot, table):
    rows, cols = multi_hot.shape
    B = rows * cols
    bpw = B // NW
    nsuper = bpw // (2 * SUPER)
    idx = multi_hot.astype(jnp.int32).reshape(NW, nsuper, SUPER, 2)
    idx = jnp.moveaxis(idx, 3, 2).reshape(NW, nsuper, 2, NCH, CHUNK)
    out2 = _make_gather(B)(idx, table)
    return out2.reshape(rows, cols, EMB)
